# Initial kernel scaffold; baseline (speedup 1.0000x reference)
import jax
import jax.numpy as jnp
from jax.experimental import pallas as pl


def kernel(inputs, cell_index, cell_mask, column_output_weights, column_output_bias):
    raise NotImplementedError("write your pallas kernel here")


# speedup vs baseline: 2.9087x; 2.9087x over previous
import jax
import jax.numpy as jnp
from jax.experimental import pallas as pl
from jax.experimental.pallas import tpu as pltpu

_B, _S, _H = 16, 4096, 768
_MAX_ROWS, _MAX_COLS = 64, 32
_NUM_CELLS = _MAX_ROWS * _MAX_COLS
_CHUNK = 512
_NCH = _S // _CHUNK
_NEG = -10000.0
_EPS = 1e-10


def _body(x_ref, idxr_ref, idxc_ref, mask_ref, w_ref, b_ref, out_ref,
          sums_ref, cnts_ref):
    c = pl.program_id(1)

    @pl.when(c == 0)
    def _init():
        sums_ref[...] = jnp.zeros_like(sums_ref)
        cnts_ref[...] = jnp.zeros_like(cnts_ref)

    x = x_ref[0]
    w = w_ref[...]
    z = jnp.sum(x * w, axis=1, keepdims=True)

    idx_row = idxr_ref[0]
    idx_col = idxc_ref[0]
    hi = idx_row >> 5
    lo = idx_col & 31

    oh_hi = (jax.lax.broadcasted_iota(jnp.int32, (_MAX_ROWS, _CHUNK), 0)
             == hi).astype(jnp.float32)
    oh_lo = (jax.lax.broadcasted_iota(jnp.int32, (_CHUNK, _MAX_COLS), 1)
             == lo).astype(jnp.float32)

    sums_ref[...] += jax.lax.dot(oh_hi, oh_lo * z,
                                 preferred_element_type=jnp.float32)
    cnts_ref[...] += jax.lax.dot(oh_hi, oh_lo,
                                 preferred_element_type=jnp.float32)

    @pl.when(c == _NCH - 1)
    def _finish():
        sums = sums_ref[...]
        cnts = cnts_ref[...]
        bias = b_ref[0, 0]
        cell_logits = jnp.where(cnts > 0.0,
                                sums / jnp.maximum(cnts, 1.0) + bias, 0.0)
        m = mask_ref[0]
        colsum = jnp.sum(cell_logits * m, axis=0, keepdims=True)
        colcnt = jnp.sum(m, axis=0, keepdims=True)
        col = colsum / (colcnt + _EPS)
        j = jax.lax.broadcasted_iota(jnp.int32, (1, _MAX_COLS), 1)
        pad = jnp.logical_and(colcnt < 0.5, j != 0)
        col = (col + _NEG * pad.astype(jnp.float32)
               + _NEG * (j == 0).astype(jnp.float32))
        out_ref[0] = col


def kernel(inputs, cell_index, cell_mask, column_output_weights,
           column_output_bias):
    idx_row = cell_index.reshape(_B * _NCH, 1, _CHUNK)
    idx_col = cell_index.reshape(_B, _S, 1)
    mask = cell_mask.reshape(_B, _MAX_ROWS, _MAX_COLS)
    w = column_output_weights.reshape(1, _H)
    b = jnp.reshape(column_output_bias, (1, 1)).astype(jnp.float32)

    grid = (_B, _NCH)
    return pl.pallas_call(
        _body,
        grid=grid,
        in_specs=[
            pl.BlockSpec((1, _CHUNK, _H), lambda b_, c: (b_, c, 0)),
            pl.BlockSpec((1, 1, _CHUNK), lambda b_, c: (b_ * _NCH + c, 0, 0)),
            pl.BlockSpec((1, _CHUNK, 1), lambda b_, c: (b_, c, 0)),
            pl.BlockSpec((1, _MAX_ROWS, _MAX_COLS), lambda b_, c: (b_, 0, 0)),
            pl.BlockSpec((1, _H), lambda b_, c: (0, 0)),
            pl.BlockSpec(memory_space=pltpu.SMEM),
        ],
        out_specs=pl.BlockSpec((1, 1, _MAX_COLS), lambda b_, c: (b_, 0, 0)),
        out_shape=jax.ShapeDtypeStruct((_B, 1, _MAX_COLS), jnp.float32),
        scratch_shapes=[
            pltpu.VMEM((_MAX_ROWS, _MAX_COLS), jnp.float32),
            pltpu.VMEM((_MAX_ROWS, _MAX_COLS), jnp.float32),
        ],
        compiler_params=pltpu.CompilerParams(
            dimension_semantics=("arbitrary", "arbitrary"),
        ),
    )(inputs, idx_row, idx_col, mask, w, b).reshape(_B, _MAX_COLS)


# speedup vs baseline: 3.9831x; 1.3694x over previous
import jax
import jax.numpy as jnp
from jax.experimental import pallas as pl
from jax.experimental.pallas import tpu as pltpu

_B, _S, _H = 16, 4096, 768
_MAX_ROWS, _MAX_COLS = 64, 32
_NUM_CELLS = _MAX_ROWS * _MAX_COLS
_CHUNK = 1024
_NCH = _S // _CHUNK
_NEG = -10000.0
_EPS = 1e-10


def _body(x_ref, idxr_ref, idxc_ref, mask_ref, w_ref, b_ref, out_ref,
          sums_ref, cnts_ref):
    c = pl.program_id(1)

    @pl.when(c == 0)
    def _init():
        sums_ref[...] = jnp.zeros_like(sums_ref)
        cnts_ref[...] = jnp.zeros_like(cnts_ref)

    x = x_ref[0]
    w = w_ref[...]
    z = jnp.sum(x * w, axis=1, keepdims=True)

    idx_row = idxr_ref[0]
    idx_col = idxc_ref[0]
    hi = idx_row >> 5
    lo = idx_col & 31

    oh_hi = (jax.lax.broadcasted_iota(jnp.int32, (_MAX_ROWS, _CHUNK), 0)
             == hi).astype(jnp.float32)
    oh_lo = (jax.lax.broadcasted_iota(jnp.int32, (_CHUNK, _MAX_COLS), 1)
             == lo).astype(jnp.float32)

    sums_ref[...] += jax.lax.dot(oh_hi, oh_lo * z,
                                 preferred_element_type=jnp.float32)
    cnts_ref[...] += jax.lax.dot(oh_hi, oh_lo,
                                 preferred_element_type=jnp.float32)

    @pl.when(c == _NCH - 1)
    def _finish():
        sums = sums_ref[...]
        cnts = cnts_ref[...]
        bias = b_ref[0, 0]
        cell_logits = jnp.where(cnts > 0.0,
                                sums / jnp.maximum(cnts, 1.0) + bias, 0.0)
        m = mask_ref[0]
        colsum = jnp.sum(cell_logits * m, axis=0, keepdims=True)
        colcnt = jnp.sum(m, axis=0, keepdims=True)
        col = colsum / (colcnt + _EPS)
        j = jax.lax.broadcasted_iota(jnp.int32, (1, _MAX_COLS), 1)
        pad = jnp.logical_and(colcnt < 0.5, j != 0)
        col = (col + _NEG * pad.astype(jnp.float32)
               + _NEG * (j == 0).astype(jnp.float32))
        out_ref[0] = col


def kernel(inputs, cell_index, cell_mask, column_output_weights,
           column_output_bias):
    idx_row = cell_index.reshape(_B * _NCH, 1, _CHUNK)
    idx_col = cell_index.reshape(_B, _S, 1)
    mask = cell_mask.reshape(_B, _MAX_ROWS, _MAX_COLS)
    w = column_output_weights.reshape(1, _H)
    b = jnp.reshape(column_output_bias, (1, 1)).astype(jnp.float32)

    grid = (_B, _NCH)
    return pl.pallas_call(
        _body,
        grid=grid,
        in_specs=[
            pl.BlockSpec((1, _CHUNK, _H), lambda b_, c: (b_, c, 0)),
            pl.BlockSpec((1, 1, _CHUNK), lambda b_, c: (b_ * _NCH + c, 0, 0)),
            pl.BlockSpec((1, _CHUNK, 1), lambda b_, c: (b_, c, 0)),
            pl.BlockSpec((1, _MAX_ROWS, _MAX_COLS), lambda b_, c: (b_, 0, 0)),
            pl.BlockSpec((1, _H), lambda b_, c: (0, 0)),
            pl.BlockSpec(memory_space=pltpu.SMEM),
        ],
        out_specs=pl.BlockSpec((1, 1, _MAX_COLS), lambda b_, c: (b_, 0, 0)),
        out_shape=jax.ShapeDtypeStruct((_B, 1, _MAX_COLS), jnp.float32),
        scratch_shapes=[
            pltpu.VMEM((_MAX_ROWS, _MAX_COLS), jnp.float32),
            pltpu.VMEM((_MAX_ROWS, _MAX_COLS), jnp.float32),
        ],
        compiler_params=pltpu.CompilerParams(
            dimension_semantics=("arbitrary", "arbitrary"),
        ),
    )(inputs, idx_row, idx_col, mask, w, b).reshape(_B, _MAX_COLS)


# speedup vs baseline: 4.8644x; 1.2212x over previous
import jax
import jax.numpy as jnp
from jax.experimental import pallas as pl
from jax.experimental.pallas import tpu as pltpu

_B, _S, _H = 16, 4096, 768
_MAX_ROWS, _MAX_COLS = 64, 32
_NUM_CELLS = _MAX_ROWS * _MAX_COLS
_CHUNK = 2048
_NCH = _S // _CHUNK
_NEG = -10000.0
_EPS = 1e-10


def _body(x_ref, idxr_ref, idxc_ref, mask_ref, w_ref, b_ref, out_ref,
          sums_ref, cnts_ref):
    c = pl.program_id(1)

    @pl.when(c == 0)
    def _init():
        sums_ref[...] = jnp.zeros_like(sums_ref)
        cnts_ref[...] = jnp.zeros_like(cnts_ref)

    x = x_ref[0]
    w = w_ref[...]
    z = jnp.sum(x * w, axis=1, keepdims=True)

    idx_row = idxr_ref[0]
    idx_col = idxc_ref[0]
    hi = idx_row >> 5
    lo = idx_col & 31

    oh_hi = (jax.lax.broadcasted_iota(jnp.int32, (_MAX_ROWS, _CHUNK), 0)
             == hi).astype(jnp.float32)
    oh_lo = (jax.lax.broadcasted_iota(jnp.int32, (_CHUNK, _MAX_COLS), 1)
             == lo).astype(jnp.float32)

    sums_ref[...] += jax.lax.dot(oh_hi, oh_lo * z,
                                 preferred_element_type=jnp.float32)
    cnts_ref[...] += jax.lax.dot(oh_hi, oh_lo,
                                 preferred_element_type=jnp.float32)

    @pl.when(c == _NCH - 1)
    def _finish():
        sums = sums_ref[...]
        cnts = cnts_ref[...]
        bias = b_ref[0, 0]
        cell_logits = jnp.where(cnts > 0.0,
                                sums / jnp.maximum(cnts, 1.0) + bias, 0.0)
        m = mask_ref[0]
        colsum = jnp.sum(cell_logits * m, axis=0, keepdims=True)
        colcnt = jnp.sum(m, axis=0, keepdims=True)
        col = colsum / (colcnt + _EPS)
        j = jax.lax.broadcasted_iota(jnp.int32, (1, _MAX_COLS), 1)
        pad = jnp.logical_and(colcnt < 0.5, j != 0)
        col = (col + _NEG * pad.astype(jnp.float32)
               + _NEG * (j == 0).astype(jnp.float32))
        out_ref[0] = col


def kernel(inputs, cell_index, cell_mask, column_output_weights,
           column_output_bias):
    idx_row = cell_index.reshape(_B * _NCH, 1, _CHUNK)
    idx_col = cell_index.reshape(_B, _S, 1)
    mask = cell_mask.reshape(_B, _MAX_ROWS, _MAX_COLS)
    w = column_output_weights.reshape(1, _H)
    b = jnp.reshape(column_output_bias, (1, 1)).astype(jnp.float32)

    grid = (_B, _NCH)
    return pl.pallas_call(
        _body,
        grid=grid,
        in_specs=[
            pl.BlockSpec((1, _CHUNK, _H), lambda b_, c: (b_, c, 0)),
            pl.BlockSpec((1, 1, _CHUNK), lambda b_, c: (b_ * _NCH + c, 0, 0)),
            pl.BlockSpec((1, _CHUNK, 1), lambda b_, c: (b_, c, 0)),
            pl.BlockSpec((1, _MAX_ROWS, _MAX_COLS), lambda b_, c: (b_, 0, 0)),
            pl.BlockSpec((1, _H), lambda b_, c: (0, 0)),
            pl.BlockSpec(memory_space=pltpu.SMEM),
        ],
        out_specs=pl.BlockSpec((1, 1, _MAX_COLS), lambda b_, c: (b_, 0, 0)),
        out_shape=jax.ShapeDtypeStruct((_B, 1, _MAX_COLS), jnp.float32),
        scratch_shapes=[
            pltpu.VMEM((_MAX_ROWS, _MAX_COLS), jnp.float32),
            pltpu.VMEM((_MAX_ROWS, _MAX_COLS), jnp.float32),
        ],
        compiler_params=pltpu.CompilerParams(
            dimension_semantics=("arbitrary", "arbitrary"),
        ),
    )(inputs, idx_row, idx_col, mask, w, b).reshape(_B, _MAX_COLS)


# speedup vs baseline: 4.9183x; 1.0111x over previous
import jax
import jax.numpy as jnp
from jax.experimental import pallas as pl
from jax.experimental.pallas import tpu as pltpu

_B, _S, _H = 16, 4096, 768
_MAX_ROWS, _MAX_COLS = 64, 32
_NUM_CELLS = _MAX_ROWS * _MAX_COLS
_CHUNK = 4096
_NCH = _S // _CHUNK
_NEG = -10000.0
_EPS = 1e-10


def _body(x_ref, idxr_ref, idxc_ref, mask_ref, w_ref, b_ref, out_ref,
          sums_ref, cnts_ref):
    c = pl.program_id(1)

    @pl.when(c == 0)
    def _init():
        sums_ref[...] = jnp.zeros_like(sums_ref)
        cnts_ref[...] = jnp.zeros_like(cnts_ref)

    x = x_ref[0]
    w = w_ref[...]
    z = jnp.sum(x * w, axis=1, keepdims=True)

    idx_row = idxr_ref[0]
    idx_col = idxc_ref[0]
    hi = idx_row >> 5
    lo = idx_col & 31

    oh_hi = (jax.lax.broadcasted_iota(jnp.int32, (_MAX_ROWS, _CHUNK), 0)
             == hi).astype(jnp.float32)
    oh_lo = (jax.lax.broadcasted_iota(jnp.int32, (_CHUNK, _MAX_COLS), 1)
             == lo).astype(jnp.float32)

    sums_ref[...] += jax.lax.dot(oh_hi, oh_lo * z,
                                 preferred_element_type=jnp.float32)
    cnts_ref[...] += jax.lax.dot(oh_hi, oh_lo,
                                 preferred_element_type=jnp.float32)

    @pl.when(c == _NCH - 1)
    def _finish():
        sums = sums_ref[...]
        cnts = cnts_ref[...]
        bias = b_ref[0, 0]
        cell_logits = jnp.where(cnts > 0.0,
                                sums / jnp.maximum(cnts, 1.0) + bias, 0.0)
        m = mask_ref[0]
        colsum = jnp.sum(cell_logits * m, axis=0, keepdims=True)
        colcnt = jnp.sum(m, axis=0, keepdims=True)
        col = colsum / (colcnt + _EPS)
        j = jax.lax.broadcasted_iota(jnp.int32, (1, _MAX_COLS), 1)
        pad = jnp.logical_and(colcnt < 0.5, j != 0)
        col = (col + _NEG * pad.astype(jnp.float32)
               + _NEG * (j == 0).astype(jnp.float32))
        out_ref[0] = col


def kernel(inputs, cell_index, cell_mask, column_output_weights,
           column_output_bias):
    idx_row = cell_index.reshape(_B * _NCH, 1, _CHUNK)
    idx_col = cell_index.reshape(_B, _S, 1)
    mask = cell_mask.reshape(_B, _MAX_ROWS, _MAX_COLS)
    w = column_output_weights.reshape(1, _H)
    b = jnp.reshape(column_output_bias, (1, 1)).astype(jnp.float32)

    grid = (_B, _NCH)
    return pl.pallas_call(
        _body,
        grid=grid,
        in_specs=[
            pl.BlockSpec((1, _CHUNK, _H), lambda b_, c: (b_, c, 0)),
            pl.BlockSpec((1, 1, _CHUNK), lambda b_, c: (b_ * _NCH + c, 0, 0)),
            pl.BlockSpec((1, _CHUNK, 1), lambda b_, c: (b_, c, 0)),
            pl.BlockSpec((1, _MAX_ROWS, _MAX_COLS), lambda b_, c: (b_, 0, 0)),
            pl.BlockSpec((1, _H), lambda b_, c: (0, 0)),
            pl.BlockSpec(memory_space=pltpu.SMEM),
        ],
        out_specs=pl.BlockSpec((1, 1, _MAX_COLS), lambda b_, c: (b_, 0, 0)),
        out_shape=jax.ShapeDtypeStruct((_B, 1, _MAX_COLS), jnp.float32),
        scratch_shapes=[
            pltpu.VMEM((_MAX_ROWS, _MAX_COLS), jnp.float32),
            pltpu.VMEM((_MAX_ROWS, _MAX_COLS), jnp.float32),
        ],
        compiler_params=pltpu.CompilerParams(
            dimension_semantics=("arbitrary", "arbitrary"),
        ),
    )(inputs, idx_row, idx_col, mask, w, b).reshape(_B, _MAX_COLS)
